# baseline (speedup 1.0000x reference)
import functools

import jax
import jax.numpy as jnp
from jax import lax
from jax.experimental import pallas as pl
from jax.experimental.pallas import tpu as pltpu
from jax.experimental.pallas import tpu_sc as plsc

D = 64
B = 4096 * 200
NC, NS = 2, 16
NW = NC * NS
BPW = B // NW
G = 128
NJ = BPW // G


@functools.cache
def _build():
  mesh = plsc.VectorSubcoreMesh(core_axis_name="c", subcore_axis_name="s")

  @functools.partial(
      pl.kernel,
      mesh=mesh,
      out_type=jax.ShapeDtypeStruct((B, D), jnp.float32),
      compiler_params=pltpu.CompilerParams(use_tc_tiling_on_sc=False),
      scratch_types=[
          pltpu.VMEM((BPW,), jnp.int32),
          pltpu.VMEM((G, D), jnp.float32),
          pltpu.SemaphoreType.DMA,
      ],
  )
  def emb(w_hbm, ids_hbm, out_hbm, idx_v, rows_v, gsem):
    wid = lax.axis_index("s") * NC + lax.axis_index("c")
    base = wid * BPW
    pltpu.sync_copy(ids_hbm.at[pl.ds(base, BPW)], idx_v)

    def body(j, carry):
      off = j * G
      desc = pltpu.make_async_copy(
          w_hbm.at[idx_v.at[pl.ds(off, G)]], rows_v, gsem)
      desc.start()
      desc.wait()
      pltpu.sync_copy(rows_v, out_hbm.at[pl.ds(base + off, G)])
      return carry

    lax.fori_loop(0, NJ, body, 0)

  return emb


def kernel(ids, W):
  ids_flat = ids.reshape(-1).astype(jnp.int32)
  out = _build()(W, ids_flat)
  return out.reshape(ids.shape + (D,))


# speedup vs baseline: 1.1135x; 1.1135x over previous
import functools

import jax
import jax.numpy as jnp
from jax import lax
from jax.experimental import pallas as pl
from jax.experimental.pallas import tpu as pltpu
from jax.experimental.pallas import tpu_sc as plsc

D = 64
B = 4096 * 200
NC, NS = 2, 16
NW = NC * NS
BPW = B // NW
G = 128
NBG = 2
GROUP = NBG * G
NG = BPW // GROUP
NBUF = 4


@functools.cache
def _build():
  mesh = plsc.VectorSubcoreMesh(core_axis_name="c", subcore_axis_name="s")

  @functools.partial(
      pl.kernel,
      mesh=mesh,
      out_type=jax.ShapeDtypeStruct((B, D), jnp.float32),
      compiler_params=pltpu.CompilerParams(use_tc_tiling_on_sc=False),
      scratch_types=[
          pltpu.VMEM((BPW,), jnp.int32),
          [pltpu.VMEM((GROUP, D), jnp.float32) for _ in range(NBUF)],
          [pltpu.SemaphoreType.DMA for _ in range(NBUF)],
          [pltpu.SemaphoreType.DMA for _ in range(NBUF)],
      ],
  )
  def emb(w_hbm, ids_hbm, out_hbm, idx_v, rows, gsem, osem):
    wid = lax.axis_index("s") * NC + lax.axis_index("c")
    base = wid * BPW
    pltpu.sync_copy(ids_hbm.at[pl.ds(base, BPW)], idx_v)

    def gathers(g, bi):
      off = g * GROUP
      return [
          pltpu.make_async_copy(
              w_hbm.at[idx_v.at[pl.ds(off + b * G, G)]],
              rows[bi].at[pl.ds(b * G, G)],
              gsem[bi],
          )
          for b in range(NBG)
      ]

    def out_copy(g, bi):
      return pltpu.make_async_copy(
          rows[bi], out_hbm.at[pl.ds(base + g * GROUP, GROUP)], osem[bi])

    def do_group(g, bi, first=False, start_next=True):
      for d in gathers(g, bi):
        d.wait()
      out_copy(g, bi).start()
      nbi = (bi + 3) % NBUF
      if not first:
        out_copy(g - 1, nbi).wait()
      if start_next:
        for d in gathers(g + 3, nbi):
          d.start()

    for g in range(3):
      for d in gathers(g, g):
        d.start()

    do_group(0, 0, first=True)
    for k in range(1, NBUF):
      do_group(k, k)

    def body(gg, carry):
      g0 = gg * NBUF
      for k in range(NBUF):
        do_group(g0 + k, k)
      return carry

    lax.fori_loop(1, NG // NBUF - 1, body, 0)

    g0 = NG - NBUF
    for k in range(NBUF):
      do_group(g0 + k, k, start_next=(g0 + k + 3 < NG))
    out_copy(NG - 1, (NG - 1) % NBUF).wait()

  return emb


def kernel(ids, W):
  ids_flat = ids.reshape(-1).astype(jnp.int32)
  out = _build()(W, ids_flat)
  return out.reshape(ids.shape + (D,))


# speedup vs baseline: 1.1163x; 1.0025x over previous
import functools

import jax
import jax.numpy as jnp
from jax import lax
from jax.experimental import pallas as pl
from jax.experimental.pallas import tpu as pltpu
from jax.experimental.pallas import tpu_sc as plsc

D = 64
NSEQ = 4096
T = 200
B = NSEQ * T
NC, NS = 2, 16
NW = NC * NS
SPW = NSEQ // NW
BPW = SPW * T
NBUF = 4


@functools.cache
def _build():
  mesh = plsc.VectorSubcoreMesh(core_axis_name="c", subcore_axis_name="s")

  @functools.partial(
      pl.kernel,
      mesh=mesh,
      out_type=jax.ShapeDtypeStruct((NSEQ, T, D), jnp.float32),
      compiler_params=pltpu.CompilerParams(use_tc_tiling_on_sc=False),
      scratch_types=[
          pltpu.VMEM((BPW,), jnp.int32),
          [pltpu.VMEM((T, D), jnp.float32) for _ in range(NBUF)],
          [pltpu.SemaphoreType.DMA for _ in range(NBUF)],
          [pltpu.SemaphoreType.DMA for _ in range(NBUF)],
      ],
  )
  def emb(w_hbm, ids_hbm, out_hbm, idx_v, rows, gsem, osem):
    wid = lax.axis_index("s") * NC + lax.axis_index("c")
    seq0 = wid * SPW
    pltpu.sync_copy(ids_hbm.at[pl.ds(wid * BPW, BPW)], idx_v)

    def gathers(g, bi):
      off = g * T
      return [
          pltpu.make_async_copy(
              w_hbm.at[idx_v.at[pl.ds(off, 128)]],
              rows[bi].at[pl.ds(0, 128)],
              gsem[bi],
          ),
          pltpu.make_async_copy(
              w_hbm.at[idx_v.at[pl.ds(off + 128, 72)]],
              rows[bi].at[pl.ds(128, 72)],
              gsem[bi],
          ),
      ]

    def out_copy(g, bi):
      return pltpu.make_async_copy(
          rows[bi], out_hbm.at[seq0 + g], osem[bi])

    def do_group(g, bi, first=False, start_next=True):
      for d in gathers(g, bi):
        d.wait()
      out_copy(g, bi).start()
      nbi = (bi + 3) % NBUF
      if not first:
        out_copy(g - 1, nbi).wait()
      if start_next:
        for d in gathers(g + 3, nbi):
          d.start()

    for g in range(3):
      for d in gathers(g, g):
        d.start()

    do_group(0, 0, first=True)
    for k in range(1, NBUF):
      do_group(k, k)

    def body(gg, carry):
      g0 = gg * NBUF
      for k in range(NBUF):
        do_group(g0 + k, k)
      return carry

    lax.fori_loop(1, SPW // NBUF - 1, body, 0)

    g0 = SPW - NBUF
    for k in range(NBUF):
      do_group(g0 + k, k, start_next=(g0 + k + 3 < SPW))
    out_copy(SPW - 1, (SPW - 1) % NBUF).wait()

  return emb


def kernel(ids, W):
  ids_flat = ids.reshape(-1).astype(jnp.int32)
  return _build()(W, ids_flat)
